# baseline (speedup 1.0000x reference)
import functools

import jax
import jax.numpy as jnp
from jax import lax
from jax.experimental import pallas as pl
from jax.experimental.pallas import tpu as pltpu
from jax.experimental.pallas import tpu_sc as plsc

B = 1024
D = 16
BN = 3072


def _make_sc_gather_t(V):
    info = plsc.get_sparse_core_info()
    NC = info.num_cores
    chunk = B // NC

    mesh = plsc.VectorSubcoreMesh(core_axis_name="c", subcore_axis_name="s")

    @functools.partial(
        pl.kernel,
        mesh=mesh,
        out_type=jax.ShapeDtypeStruct((D, B), jnp.float32),
        scratch_types=[
            pltpu.VMEM((V,), jnp.float32),
            pltpu.VMEM((chunk,), jnp.int32),
            pltpu.VMEM((chunk,), jnp.float32),
        ],
        compiler_params=pltpu.CompilerParams(needs_layout_passes=False),
    )
    def gather(table_hbm, idx_hbm, out_hbm, row_v, idx_v, vals_v):
        c = lax.axis_index("c")
        s = lax.axis_index("s")
        base = c * chunk
        pltpu.sync_copy(table_hbm.at[s], row_v)
        pltpu.sync_copy(idx_hbm.at[pl.ds(base, chunk)], idx_v)

        def body(i, carry):
            ids = idx_v[pl.ds(i * 16, 16)]
            vals_v[pl.ds(i * 16, 16)] = plsc.load_gather(row_v, [ids])
            return carry

        lax.fori_loop(0, chunk // 16, body, 0)
        pltpu.sync_copy(vals_v, out_hbm.at[s, pl.ds(base, chunk)])

    return gather


def _proj_t_block(w_ref, b_ref, emb_ref, out_ref):
    w_aug = jnp.concatenate([w_ref[...], b_ref[...]], axis=0)
    ones = jnp.ones((1, B), jnp.float32)
    emb_aug = jnp.concatenate([emb_ref[...], ones], axis=0)
    out_ref[...] = lax.dot_general(
        w_aug,
        emb_aug,
        dimension_numbers=(((0,), (0,)), ((), ())),
        preferred_element_type=jnp.float32,
    )


def kernel(target_word, emb_table, lin_w, lin_b):
    V = emb_table.shape[0]
    idx = target_word.astype(jnp.int32)

    table_t = emb_table.T
    emb_t = _make_sc_gather_t(V)(table_t, idx)

    w_t = lin_w.T
    grid = pl.cdiv(V, BN)
    out_t = pl.pallas_call(
        _proj_t_block,
        grid=(grid,),
        in_specs=[
            pl.BlockSpec((D, BN), lambda j: (0, j)),
            pl.BlockSpec((1, BN), lambda j: (0, j)),
            pl.BlockSpec((D, B), lambda j: (0, 0)),
        ],
        out_specs=pl.BlockSpec((BN, B), lambda j: (j, 0)),
        out_shape=jax.ShapeDtypeStruct((V, B), jnp.float32),
    )(w_t, lin_b.reshape(1, V), emb_t)
    return out_t.T


# speedup vs baseline: 1.0016x; 1.0016x over previous
import functools

import jax
import jax.numpy as jnp
from jax import lax
from jax.experimental import pallas as pl
from jax.experimental.pallas import tpu as pltpu
from jax.experimental.pallas import tpu_sc as plsc

B = 1024
D = 16
BN = 2560


def _make_sc_gather_t(V):
    info = plsc.get_sparse_core_info()
    NC = info.num_cores
    chunk = B // NC

    mesh = plsc.VectorSubcoreMesh(core_axis_name="c", subcore_axis_name="s")

    @functools.partial(
        pl.kernel,
        mesh=mesh,
        out_type=jax.ShapeDtypeStruct((D, B), jnp.float32),
        scratch_types=[
            pltpu.VMEM((V,), jnp.float32),
            pltpu.VMEM((chunk,), jnp.int32),
            pltpu.VMEM((chunk,), jnp.float32),
        ],
        compiler_params=pltpu.CompilerParams(
            needs_layout_passes=False, skip_device_barrier=True
        ),
    )
    def gather(table_hbm, idx_hbm, out_hbm, row_v, idx_v, vals_v):
        c = lax.axis_index("c")
        s = lax.axis_index("s")
        base = c * chunk
        pltpu.sync_copy(table_hbm.at[s], row_v)
        pltpu.sync_copy(idx_hbm.at[pl.ds(base, chunk)], idx_v)

        def body(i, carry):
            ids = idx_v[pl.ds(i * 16, 16)]
            vals_v[pl.ds(i * 16, 16)] = plsc.load_gather(row_v, [ids])
            return carry

        lax.fori_loop(0, chunk // 16, body, 0)
        pltpu.sync_copy(vals_v, out_hbm.at[s, pl.ds(base, chunk)])

    return gather


def _proj_t_block(w_ref, b_ref, emb_ref, out_ref):
    w_aug = jnp.concatenate([w_ref[...], b_ref[...]], axis=0)
    ones = jnp.ones((1, B), jnp.float32)
    emb_aug = jnp.concatenate([emb_ref[...], ones], axis=0)
    out_ref[...] = lax.dot_general(
        w_aug,
        emb_aug,
        dimension_numbers=(((0,), (0,)), ((), ())),
        preferred_element_type=jnp.float32,
    )


def kernel(target_word, emb_table, lin_w, lin_b):
    V = emb_table.shape[0]
    idx = target_word.astype(jnp.int32)

    table_t = emb_table.T
    emb_t = _make_sc_gather_t(V)(table_t, idx)

    w_t = lin_w.T
    grid = pl.cdiv(V, BN)
    out_t = pl.pallas_call(
        _proj_t_block,
        grid=(grid,),
        in_specs=[
            pl.BlockSpec((D, BN), lambda j: (0, j)),
            pl.BlockSpec((1, BN), lambda j: (0, j)),
            pl.BlockSpec((D, B), lambda j: (0, 0)),
        ],
        out_specs=pl.BlockSpec((BN, B), lambda j: (j, 0)),
        out_shape=jax.ShapeDtypeStruct((V, B), jnp.float32),
    )(w_t, lin_b.reshape(1, V), emb_t)
    return out_t.T


# speedup vs baseline: 1.0052x; 1.0035x over previous
import functools

import jax
import jax.numpy as jnp
from jax import lax
from jax.experimental import pallas as pl
from jax.experimental.pallas import tpu as pltpu
from jax.experimental.pallas import tpu_sc as plsc

B = 1024
D = 16
BN = 2560


def _make_sc_gather_t(V):
    info = plsc.get_sparse_core_info()
    NC = info.num_cores
    rows_per_core = D // NC

    mesh = plsc.VectorSubcoreMesh(core_axis_name="c", subcore_axis_name="s")

    @functools.partial(
        pl.kernel,
        mesh=mesh,
        out_type=jax.ShapeDtypeStruct((D, B), jnp.float32),
        scratch_types=[
            pltpu.VMEM((V,), jnp.float32),
            pltpu.VMEM((B,), jnp.int32),
            pltpu.VMEM((B,), jnp.float32),
        ],
        compiler_params=pltpu.CompilerParams(needs_layout_passes=False),
    )
    def gather(table_hbm, idx_hbm, out_hbm, row_v, idx_v, vals_v):
        c = lax.axis_index("c")
        s = lax.axis_index("s")

        @pl.when(s < rows_per_core)
        def _():
            r = c * rows_per_core + s
            pltpu.sync_copy(table_hbm.at[r], row_v)
            pltpu.sync_copy(idx_hbm, idx_v)

            def body(i, carry):
                ids = idx_v[pl.ds(i * 16, 16)]
                vals_v[pl.ds(i * 16, 16)] = plsc.load_gather(row_v, [ids])
                return carry

            lax.fori_loop(0, B // 16, body, 0)
            pltpu.sync_copy(vals_v, out_hbm.at[r])

    return gather


def _proj_t_block(w_ref, b_ref, emb_ref, out_ref):
    w_aug = jnp.concatenate([w_ref[...], b_ref[...]], axis=0)
    ones = jnp.ones((1, B), jnp.float32)
    emb_aug = jnp.concatenate([emb_ref[...], ones], axis=0)
    out_ref[...] = lax.dot_general(
        w_aug,
        emb_aug,
        dimension_numbers=(((0,), (0,)), ((), ())),
        preferred_element_type=jnp.float32,
    )


def kernel(target_word, emb_table, lin_w, lin_b):
    V = emb_table.shape[0]
    idx = target_word.astype(jnp.int32)

    table_t = emb_table.T
    emb_t = _make_sc_gather_t(V)(table_t, idx)

    w_t = lin_w.T
    grid = pl.cdiv(V, BN)
    out_t = pl.pallas_call(
        _proj_t_block,
        grid=(grid,),
        in_specs=[
            pl.BlockSpec((D, BN), lambda j: (0, j)),
            pl.BlockSpec((1, BN), lambda j: (0, j)),
            pl.BlockSpec((D, B), lambda j: (0, 0)),
        ],
        out_specs=pl.BlockSpec((BN, B), lambda j: (j, 0)),
        out_shape=jax.ShapeDtypeStruct((V, B), jnp.float32),
    )(w_t, lin_b.reshape(1, V), emb_t)
    return out_t.T
